# Initial kernel scaffold; baseline (speedup 1.0000x reference)
import jax
import jax.numpy as jnp
from jax.experimental import pallas as pl


def kernel(x, edge_index, W0, aS0, aD0, b0, W1, aS1, aD1, b1, W2, aS2, aD2, b2):
    raise NotImplementedError("write your pallas kernel here")


# speedup vs baseline: 36.5158x; 36.5158x over previous
import functools

import jax
import jax.numpy as jnp
from jax import lax
from jax.experimental import pallas as pl
from jax.experimental.pallas import tpu as pltpu
from jax.experimental.pallas import tpu_sc as plsc

N_NODES = 10000
N_EDGES = 320000
CH = 128
ROW = 144
BR = 2000
NC = 2
NS = 16
NW = NC * NS
EPW = N_EDGES // NW
EB = 80
NB = EPW // EB
ZC = 16
NCHUNK = N_NODES // ZC


def _dense0_body(x_ref, w_ref, ase_ref, ade_ref, hx_ref, ad_ref):
    h = jnp.dot(x_ref[...], w_ref[...], preferred_element_type=jnp.float32)
    hx_ref[:, :CH] = h
    hx_ref[:, CH:] = jnp.dot(h, ase_ref[...], preferred_element_type=jnp.float32)
    ad_ref[...] = jnp.dot(h, ade_ref[...], preferred_element_type=jnp.float32)


def _dense0(x, W, AsE, AdE):
    return pl.pallas_call(
        _dense0_body,
        grid=(N_NODES // BR,),
        in_specs=[
            pl.BlockSpec((BR, CH), lambda i: (i, 0)),
            pl.BlockSpec((CH, CH), lambda i: (0, 0)),
            pl.BlockSpec((CH, 16), lambda i: (0, 0)),
            pl.BlockSpec((CH, 16), lambda i: (0, 0)),
        ],
        out_specs=[
            pl.BlockSpec((BR, ROW), lambda i: (i, 0)),
            pl.BlockSpec((BR, 16), lambda i: (i, 0)),
        ],
        out_shape=[
            jax.ShapeDtypeStruct((N_NODES, ROW), jnp.float32),
            jax.ShapeDtypeStruct((N_NODES, 16), jnp.float32),
        ],
    )(x, W, AsE, AdE)


def _combine_body(acc_ref, b_ref, exp_ref, w_ref, ase_ref, ade_ref, hx_ref, ad_ref):
    s = acc_ref[0] + acc_ref[1]
    num = s[:, :CH]
    dexp = jnp.dot(s[:, CH:], exp_ref[...], preferred_element_type=jnp.float32) + 1e-16
    xv = jnp.maximum(num / dexp + b_ref[...], 0.0)
    h = jnp.dot(xv, w_ref[...], preferred_element_type=jnp.float32)
    hx_ref[:, :CH] = h
    hx_ref[:, CH:] = jnp.dot(h, ase_ref[...], preferred_element_type=jnp.float32)
    ad_ref[...] = jnp.dot(h, ade_ref[...], preferred_element_type=jnp.float32)


def _combine(accden, b, Exp, W, AsE, AdE):
    return pl.pallas_call(
        _combine_body,
        grid=(N_NODES // BR,),
        in_specs=[
            pl.BlockSpec((NC, BR, ROW), lambda i: (0, i, 0)),
            pl.BlockSpec((1, CH), lambda i: (0, 0)),
            pl.BlockSpec((16, CH), lambda i: (0, 0)),
            pl.BlockSpec((CH, CH), lambda i: (0, 0)),
            pl.BlockSpec((CH, 16), lambda i: (0, 0)),
            pl.BlockSpec((CH, 16), lambda i: (0, 0)),
        ],
        out_specs=[
            pl.BlockSpec((BR, ROW), lambda i: (i, 0)),
            pl.BlockSpec((BR, 16), lambda i: (i, 0)),
        ],
        out_shape=[
            jax.ShapeDtypeStruct((N_NODES, ROW), jnp.float32),
            jax.ShapeDtypeStruct((N_NODES, 16), jnp.float32),
        ],
    )(accden, b, Exp, W, AsE, AdE)


def _final_body(acc_ref, b_ref, exp_ref, out_ref):
    s = acc_ref[0] + acc_ref[1]
    dexp = jnp.dot(s[:, CH:], exp_ref[...], preferred_element_type=jnp.float32) + 1e-16
    out_ref[...] = s[:, :CH] / dexp + b_ref[...]


def _final(accden, b, Exp):
    return pl.pallas_call(
        _final_body,
        grid=(N_NODES // BR,),
        in_specs=[
            pl.BlockSpec((NC, BR, ROW), lambda i: (0, i, 0)),
            pl.BlockSpec((1, CH), lambda i: (0, 0)),
            pl.BlockSpec((16, CH), lambda i: (0, 0)),
        ],
        out_specs=pl.BlockSpec((BR, CH), lambda i: (i, 0)),
        out_shape=jax.ShapeDtypeStruct((N_NODES, CH), jnp.float32),
    )(accden, b, Exp)


def _make_edge(headlane):
    mesh = plsc.VectorSubcoreMesh(core_axis_name="c", subcore_axis_name="s")

    @functools.partial(
        pl.kernel,
        out_type=jax.ShapeDtypeStruct((NC, N_NODES, ROW), jnp.float32),
        mesh=mesh,
        scratch_types=[
            pltpu.VMEM((EB,), jnp.int32),
            pltpu.VMEM((EB,), jnp.int32),
            pltpu.VMEM((EB, ROW), jnp.float32),
            pltpu.VMEM((EB, 16), jnp.float32),
            pltpu.VMEM((ZC, ROW), jnp.float32),
            pltpu.VMEM_SHARED((N_NODES, ROW), jnp.float32),
            pltpu.SemaphoreType.DMA,
            pltpu.SemaphoreType.DMA,
        ],
        compiler_params=pltpu.CompilerParams(use_tc_tiling_on_sc=False),
    )
    def k(hx, ad, srcr, dstr, out, idxs, idxd, hrows, brows, zbuf, accsh, sem1, sem2):
        cid = lax.axis_index("c")
        sid = lax.axis_index("s")
        wid = cid * NS + sid
        zv = jnp.zeros((16,), jnp.float32)
        for r in range(ZC):
            for j in range(ROW // 16):
                zbuf[r, pl.ds(16 * j, 16)] = zv

        def zchunk(t, carry):
            c = sid + NS * t

            @pl.when(c < NCHUNK)
            def _():
                pltpu.sync_copy(zbuf, accsh.at[pl.ds(c * ZC, ZC)])

            return carry

        lax.fori_loop(0, NCHUNK // NS + 1, zchunk, 0)
        plsc.subcore_barrier()
        base0 = pl.multiple_of(wid * EPW, 8)

        def blk(bi, carry):
            base = pl.multiple_of(base0 + bi * EB, 8)
            pltpu.sync_copy(srcr.at[pl.ds(base, EB)], idxs)
            pltpu.sync_copy(dstr.at[pl.ds(base, EB)], idxd)
            cp1 = pltpu.async_copy(hx.at[idxs], hrows, sem1)
            cp2 = pltpu.async_copy(ad.at[idxd], brows, sem2)
            cp1.wait()
            cp2.wait()

            def edge(e, c2):
                av = hrows[e, pl.ds(CH, 16)]
                bv = brows[e, :]
                es = av + bv
                es = jnp.where(es >= 0.0, es, es * 0.2)
                w = jnp.exp(es)
                hrows[e, pl.ds(CH, 16)] = w
                for j in range(8):
                    ws = w[headlane[j]]
                    hrows[e, pl.ds(16 * j, 16)] = hrows[e, pl.ds(16 * j, 16)] * ws
                return c2

            lax.fori_loop(0, EB, edge, 0)
            pltpu.sync_copy(hrows, accsh.at[idxd], add=True)
            return carry

        lax.fori_loop(0, NB, blk, 0)
        plsc.subcore_barrier()

        def rchunk(t, carry):
            c = sid + NS * t

            @pl.when(c < NCHUNK)
            def _():
                pltpu.sync_copy(accsh.at[pl.ds(c * ZC, ZC)], zbuf)
                pltpu.sync_copy(zbuf, out.at[cid, pl.ds(c * ZC, ZC)])

            return carry

        lax.fori_loop(0, NCHUNK // NS + 1, rchunk, 0)

    return k


_edge4 = _make_edge((0, 0, 1, 1, 2, 2, 3, 3))
_edge1 = _make_edge((0,) * 8)


def _expand_a(a):
    H, C = a.shape
    eye = jnp.eye(16, dtype=a.dtype)[:H][:, None, :]
    return (a[:, :, None] * eye).reshape(H * C, 16)


def _expand_mat(C):
    return (jnp.arange(16)[:, None] == (jnp.arange(CH)[None, :] // C)
            ).astype(jnp.float32)


def kernel(x, edge_index, W0, aS0, aD0, b0, W1, aS1, aD1, b1,
           W2, aS2, aD2, b2):
    src = edge_index[0]
    dst = edge_index[1]
    exp4 = _expand_mat(32)
    exp1 = _expand_mat(128)

    hx, ad = _dense0(x, W0, _expand_a(aS0), _expand_a(aD0))
    accden = _edge4(hx, ad, src, dst)
    hx, ad = _combine(accden, b0.reshape(1, CH), exp4, W1,
                      _expand_a(aS1), _expand_a(aD1))
    accden = _edge4(hx, ad, src, dst)
    hx, ad = _combine(accden, b1.reshape(1, CH), exp4, W2,
                      _expand_a(aS2), _expand_a(aD2))
    accden = _edge1(hx, ad, src, dst)
    return _final(accden, b2.reshape(1, CH), exp1)
